# baseline (speedup 1.0000x reference)
import functools

import jax
import jax.numpy as jnp
from jax import lax
from jax.experimental import pallas as pl
from jax.experimental.pallas import tpu as pltpu
from jax.experimental.pallas import tpu_sc as plsc

_NUM_CORES = 2
_NUM_SUBCORES = 16
_CHUNK_ROWS = 16
_NBUF = 7


def _positional_rows(pos_table, seq_len):
    d_model = pos_table.shape[1]
    num_workers = _NUM_CORES * _NUM_SUBCORES
    rows_per_w = seq_len // num_workers
    assert rows_per_w * num_workers == seq_len
    assert rows_per_w % _CHUNK_ROWS == 0
    nchunks = rows_per_w // _CHUNK_ROWS

    mesh = plsc.VectorSubcoreMesh(core_axis_name="c", subcore_axis_name="s")

    @functools.partial(
        pl.kernel,
        out_type=jax.ShapeDtypeStruct((seq_len, d_model), pos_table.dtype),
        mesh=mesh,
        scratch_types=[
            pltpu.VMEM((_NBUF, _CHUNK_ROWS, d_model), pos_table.dtype),
            pltpu.SemaphoreType.DMA((_NBUF,)),
            pltpu.SemaphoreType.DMA((_NBUF,)),
        ],
    )
    def copy_rows(table_hbm, out_hbm, buf, sems_in, sems_out):
        wid = lax.axis_index("s") * _NUM_CORES + lax.axis_index("c")
        base = wid * rows_per_w

        def start_in(j):
            b = j % _NBUF
            return pltpu.async_copy(
                table_hbm.at[pl.ds(base + j * _CHUNK_ROWS, _CHUNK_ROWS)],
                buf.at[b],
                sems_in.at[b],
            )

        def start_out(j):
            b = j % _NBUF
            return pltpu.async_copy(
                buf.at[b],
                out_hbm.at[pl.ds(base + j * _CHUNK_ROWS, _CHUNK_ROWS)],
                sems_out.at[b],
            )

        in_flight = [start_in(j) for j in range(min(_NBUF, nchunks))]
        out_flight = [None] * nchunks
        for j in range(nchunks):
            in_flight[j].wait()
            out_flight[j] = start_out(j)
            if j + _NBUF < nchunks:
                out_flight[j].wait()
                in_flight.append(start_in(j + _NBUF))
        for j in range(max(0, nchunks - _NBUF), nchunks):
            out_flight[j].wait()

    return copy_rows(pos_table)


def kernel(x, pos_table):
    seq_len = x.shape[1]
    return _positional_rows(pos_table, seq_len)[None]


# speedup vs baseline: 1.0083x; 1.0083x over previous
import functools

import jax
import jax.numpy as jnp
from jax import lax
from jax.experimental import pallas as pl
from jax.experimental.pallas import tpu as pltpu
from jax.experimental.pallas import tpu_sc as plsc

_NUM_CORES = 2
_NUM_SUBCORES = 16
_CHUNK_ROWS = 64
_NBUF = 2


def _positional_rows(pos_table, seq_len):
    d_model = pos_table.shape[1]
    num_workers = _NUM_CORES * _NUM_SUBCORES
    rows_per_w = seq_len // num_workers
    assert rows_per_w * num_workers == seq_len
    assert rows_per_w % _CHUNK_ROWS == 0
    nchunks = rows_per_w // _CHUNK_ROWS

    mesh = plsc.VectorSubcoreMesh(core_axis_name="c", subcore_axis_name="s")

    @functools.partial(
        pl.kernel,
        out_type=jax.ShapeDtypeStruct((seq_len, d_model), pos_table.dtype),
        mesh=mesh,
        scratch_types=[
            pltpu.VMEM((_NBUF, _CHUNK_ROWS, d_model), pos_table.dtype),
            pltpu.SemaphoreType.DMA((_NBUF,)),
            pltpu.SemaphoreType.DMA((_NBUF,)),
        ],
    )
    def copy_rows(table_hbm, out_hbm, buf, sems_in, sems_out):
        wid = lax.axis_index("s") * _NUM_CORES + lax.axis_index("c")
        base = wid * rows_per_w

        def start_in(j):
            b = j % _NBUF
            return pltpu.async_copy(
                table_hbm.at[pl.ds(base + j * _CHUNK_ROWS, _CHUNK_ROWS)],
                buf.at[b],
                sems_in.at[b],
            )

        def start_out(j):
            b = j % _NBUF
            return pltpu.async_copy(
                buf.at[b],
                out_hbm.at[pl.ds(base + j * _CHUNK_ROWS, _CHUNK_ROWS)],
                sems_out.at[b],
            )

        in_flight = [start_in(j) for j in range(min(_NBUF, nchunks))]
        out_flight = [None] * nchunks
        for j in range(nchunks):
            in_flight[j].wait()
            out_flight[j] = start_out(j)
            if j + _NBUF < nchunks:
                out_flight[j].wait()
                in_flight.append(start_in(j + _NBUF))
        for j in range(max(0, nchunks - _NBUF), nchunks):
            out_flight[j].wait()

    return copy_rows(pos_table)


def kernel(x, pos_table):
    seq_len = x.shape[1]
    return _positional_rows(pos_table, seq_len)[None]
